# baseline (speedup 1.0000x reference)
import jax
import jax.numpy as jnp
from jax import lax
from jax.experimental import pallas as pl
from jax.experimental.pallas import tpu as pltpu
from jax.experimental.pallas import tpu_sc as plsc

N = 10000
E = 320000
D_IN = 128
D_OUT = 40

NC = 2
NS = 16
NW = NC * NS
L = 16

CHUNK = 128
N_PAD = 10240
DUMP = N_PAD - 1
NCH_PAD = 2560
E_PAD = NCH_PAD * CHUNK
CPW = NCH_PAD // NW
CPT = NCH_PAD // NS
ROWS_PER_TILE = N_PAD // NS
IBLK = 16
DEG_W = 16

_MESH = plsc.VectorSubcoreMesh(
    core_axis_name="c", subcore_axis_name="s", num_cores=NC, num_subcores=NS
)


def _deg_body(edge_hbm, deg_hbm, idx_v, ones_v, zbuf_v, dacc, sem):
    cid = lax.axis_index("c")
    tid = lax.axis_index("s")

    def init_loop(i, carry):
        zbuf_v[pl.ds(i * L, L)] = jnp.zeros((L,), jnp.float32)
        return carry

    lax.fori_loop(0, ROWS_PER_TILE // L, init_loop, 0)

    def ones_loop(i, carry):
        ones_v[pl.ds(i * L, L)] = jnp.ones((L,), jnp.float32)
        return carry

    lax.fori_loop(0, CHUNK // L, ones_loop, 0)

    pltpu.sync_copy(edge_hbm.at[cid, pl.ds(tid * CPT, CPT)], idx_v)

    pltpu.sync_copy(zbuf_v, dacc.at[pl.ds(tid * ROWS_PER_TILE, ROWS_PER_TILE)])
    plsc.subcore_barrier()

    def body(j, carry):
        pltpu.sync_copy(ones_v, dacc.at[idx_v.at[j]], add=True)
        return carry

    lax.fori_loop(0, CPT, body, 0)
    plsc.subcore_barrier()

    s = tid * ROWS_PER_TILE
    pltpu.sync_copy(
        dacc.at[pl.ds(s, ROWS_PER_TILE)],
        deg_hbm.at[cid, pl.ds(s, ROWS_PER_TILE)],
    )


_deg = pl.kernel(
    _deg_body,
    out_type=jax.ShapeDtypeStruct((NC, N_PAD), jnp.float32),
    mesh=_MESH,
    scratch_types=[
        pltpu.VMEM((CPT, CHUNK), jnp.int32),
        pltpu.VMEM((CHUNK,), jnp.float32),
        pltpu.VMEM((ROWS_PER_TILE,), jnp.float32),
        pltpu.VMEM_SHARED((N_PAD,), jnp.float32),
        pltpu.SemaphoreType.DMA,
    ],
)


def _agg_body(h_hbm, edge_hbm, accp_hbm, sidx_v, didx_v, r0, r1, acc, s0, s1):
    cid = lax.axis_index("c")
    tid = lax.axis_index("s")
    wid = cid * NS + tid

    def zloop(i, carry):
        r0[i // (D_IN // L), pl.ds((i % (D_IN // L)) * L, L)] = jnp.zeros(
            (L,), jnp.float32
        )
        return carry

    lax.fori_loop(0, CHUNK * (D_IN // L), zloop, 0)

    def zcopy(k, carry):
        pltpu.sync_copy(r0, acc.at[pl.ds(tid * ROWS_PER_TILE + k * CHUNK, CHUNK)])
        return carry

    lax.fori_loop(0, ROWS_PER_TILE // CHUNK, zcopy, 0)

    pltpu.sync_copy(edge_hbm.at[1, pl.ds(wid * CPW, CPW)], didx_v)
    plsc.subcore_barrier()

    def block(ib, carry):
        pltpu.sync_copy(edge_hbm.at[0, pl.ds(wid * CPW + ib * IBLK, IBLK)], sidx_v)
        pltpu.async_copy(h_hbm.at[sidx_v.at[0]], r0, s0)

        def pair(k, carry2):
            c = ib * IBLK + 2 * k
            pltpu.async_copy(h_hbm.at[sidx_v.at[2 * k + 1]], r1, s1)
            pltpu.make_async_copy(h_hbm.at[sidx_v.at[0]], r0, s0).wait()
            pltpu.sync_copy(r0, acc.at[didx_v.at[c]], add=True)

            @pl.when(2 * k + 2 < IBLK)
            def _():
                pltpu.async_copy(h_hbm.at[sidx_v.at[2 * k + 2]], r0, s0)

            pltpu.make_async_copy(h_hbm.at[sidx_v.at[0]], r1, s1).wait()
            pltpu.sync_copy(r1, acc.at[didx_v.at[c + 1]], add=True)
            return carry2

        lax.fori_loop(0, IBLK // 2, pair, 0)
        return carry

    lax.fori_loop(0, CPW // IBLK, block, 0)
    plsc.subcore_barrier()

    def wb(k, carry):
        r = tid * ROWS_PER_TILE + k * CHUNK
        pltpu.sync_copy(acc.at[pl.ds(r, CHUNK)], accp_hbm.at[cid, pl.ds(r, CHUNK)])
        return carry

    lax.fori_loop(0, ROWS_PER_TILE // CHUNK, wb, 0)


_agg = pl.kernel(
    _agg_body,
    out_type=jax.ShapeDtypeStruct((NC, N_PAD, D_IN), jnp.float32),
    mesh=_MESH,
    scratch_types=[
        pltpu.VMEM((IBLK, CHUNK), jnp.int32),
        pltpu.VMEM((CPW, CHUNK), jnp.int32),
        pltpu.VMEM((CHUNK, D_IN), jnp.float32),
        pltpu.VMEM((CHUNK, D_IN), jnp.float32),
        pltpu.VMEM_SHARED((N_PAD, D_IN), jnp.float32),
        pltpu.SemaphoreType.DMA,
        pltpu.SemaphoreType.DMA,
    ],
)


ROW_BLK = 1024


def _scale_body(x_ref, d_ref, o_ref):
    o_ref[...] = x_ref[...] * lax.rsqrt(jnp.maximum(d_ref[...], 1.0))


_scale = pl.pallas_call(
    _scale_body,
    out_shape=jax.ShapeDtypeStruct((N_PAD, D_IN), jnp.float32),
    grid=(N_PAD // ROW_BLK,),
    in_specs=[
        pl.BlockSpec((ROW_BLK, D_IN), lambda i: (i, 0)),
        pl.BlockSpec((ROW_BLK, 1), lambda i: (i, 0)),
    ],
    out_specs=pl.BlockSpec((ROW_BLK, D_IN), lambda i: (i, 0)),
)


HEAD_BLK = 1000


def _head_body(a_ref, d_ref, w1_ref, b1_ref, w2_ref, b2_ref, o_ref):
    a = a_ref[0] + a_ref[1]
    a = a * lax.rsqrt(jnp.maximum(d_ref[...], 1.0))
    h2 = jnp.dot(a, w1_ref[...], preferred_element_type=jnp.float32) + b1_ref[...]
    h2 = jnp.maximum(h2, 0.0)
    o_ref[...] = (
        jnp.dot(h2, w2_ref[...], preferred_element_type=jnp.float32) + b2_ref[...]
    )


_head = pl.pallas_call(
    _head_body,
    out_shape=jax.ShapeDtypeStruct((N, D_OUT), jnp.float32),
    grid=(N // HEAD_BLK,),
    in_specs=[
        pl.BlockSpec((NC, HEAD_BLK, D_IN), lambda i: (0, i, 0)),
        pl.BlockSpec((HEAD_BLK, 1), lambda i: (i, 0)),
        pl.BlockSpec((D_IN, D_IN), lambda i: (0, 0)),
        pl.BlockSpec((1, D_IN), lambda i: (0, 0)),
        pl.BlockSpec((D_IN, D_OUT), lambda i: (0, 0)),
        pl.BlockSpec((1, D_OUT), lambda i: (0, 0)),
    ],
    out_specs=pl.BlockSpec((HEAD_BLK, D_OUT), lambda i: (i, 0)),
)


@jax.jit
def kernel(n_feat, edge_index, W_gcn, b_gcn, W_lin, b_lin):
    ei = jnp.concatenate(
        [edge_index, jnp.full((2, E_PAD - E), DUMP, jnp.int32)], axis=1
    ).reshape(2, NCH_PAD, CHUNK)
    x = jnp.concatenate(
        [n_feat, jnp.zeros((N_PAD - N, D_IN), jnp.float32)], axis=0
    )
    deg = _deg(ei)
    h = _scale(x, deg[0].reshape(N_PAD, 1))
    accp = _agg(h, ei)[:, :N]
    out = _head(
        accp,
        deg[1, :N].reshape(N, 1),
        W_gcn,
        b_gcn.reshape(1, D_IN),
        W_lin,
        b_lin.reshape(1, D_OUT),
    )
    return out


# speedup vs baseline: 3.1515x; 3.1515x over previous
import jax
import jax.numpy as jnp
from jax import lax
from jax.experimental import pallas as pl
from jax.experimental.pallas import tpu as pltpu
from jax.experimental.pallas import tpu_sc as plsc

N = 10000
E = 320000
D_IN = 128
D_OUT = 40

NC = 2
NS = 16
NW = NC * NS
L = 16

CHUNK = 128
N_PAD = 10240
DUMP = N_PAD - 1
NCH_PAD = 2560
E_PAD = NCH_PAD * CHUNK
CPW = NCH_PAD // NW
CPT = NCH_PAD // NS
ROWS_PER_TILE = N_PAD // NS
IBLK = 16
DEG_W = 16

_MESH = plsc.VectorSubcoreMesh(
    core_axis_name="c", subcore_axis_name="s", num_cores=NC, num_subcores=NS
)


def _deg_body(edge_hbm, deg_hbm, idx_v, ones_v, zbuf_v, dacc, sem):
    cid = lax.axis_index("c")
    tid = lax.axis_index("s")

    def init_loop(i, carry):
        zbuf_v[pl.ds(i * L, L)] = jnp.zeros((L,), jnp.float32)
        return carry

    lax.fori_loop(0, ROWS_PER_TILE // L, init_loop, 0)

    def ones_loop(i, carry):
        ones_v[pl.ds(i * L, L)] = jnp.ones((L,), jnp.float32)
        return carry

    lax.fori_loop(0, CHUNK // L, ones_loop, 0)

    pltpu.sync_copy(edge_hbm.at[cid, pl.ds(tid * CPT, CPT)], idx_v)

    pltpu.sync_copy(zbuf_v, dacc.at[pl.ds(tid * ROWS_PER_TILE, ROWS_PER_TILE)])
    plsc.subcore_barrier()

    def body(j, carry):
        pltpu.sync_copy(ones_v, dacc.at[idx_v.at[j]], add=True)
        return carry

    lax.fori_loop(0, CPT, body, 0)
    plsc.subcore_barrier()

    s = tid * ROWS_PER_TILE
    pltpu.sync_copy(
        dacc.at[pl.ds(s, ROWS_PER_TILE)],
        deg_hbm.at[cid, pl.ds(s, ROWS_PER_TILE)],
    )


_deg = pl.kernel(
    _deg_body,
    out_type=jax.ShapeDtypeStruct((NC, N_PAD), jnp.float32),
    mesh=_MESH,
    scratch_types=[
        pltpu.VMEM((CPT, CHUNK), jnp.int32),
        pltpu.VMEM((CHUNK,), jnp.float32),
        pltpu.VMEM((ROWS_PER_TILE,), jnp.float32),
        pltpu.VMEM_SHARED((N_PAD,), jnp.float32),
        pltpu.SemaphoreType.DMA,
    ],
)


def _agg_body(h_hbm, edge_hbm, accp_hbm, sidx_v, didx_v, r0, r1, acc, s0, s1):
    cid = lax.axis_index("c")
    tid = lax.axis_index("s")
    wid = cid * NS + tid

    def zloop(i, carry):
        r0[i // (D_IN // L), pl.ds((i % (D_IN // L)) * L, L)] = jnp.zeros(
            (L,), jnp.float32
        )
        return carry

    lax.fori_loop(0, CHUNK * (D_IN // L), zloop, 0)

    def zcopy(k, carry):
        pltpu.sync_copy(r0, acc.at[pl.ds(tid * ROWS_PER_TILE + k * CHUNK, CHUNK)])
        return carry

    lax.fori_loop(0, ROWS_PER_TILE // CHUNK, zcopy, 0)

    pltpu.sync_copy(edge_hbm.at[1, pl.ds(wid * CPW, CPW)], didx_v)
    plsc.subcore_barrier()

    def block(ib, carry):
        pltpu.sync_copy(edge_hbm.at[0, pl.ds(wid * CPW + ib * IBLK, IBLK)], sidx_v)
        pltpu.async_copy(h_hbm.at[sidx_v.at[0]], r0, s0)

        def pair(k, carry2):
            c = ib * IBLK + 2 * k
            pltpu.async_copy(h_hbm.at[sidx_v.at[2 * k + 1]], r1, s1)
            pltpu.make_async_copy(h_hbm.at[sidx_v.at[0]], r0, s0).wait()
            pltpu.sync_copy(r0, acc.at[didx_v.at[c]], add=True)

            @pl.when(2 * k + 2 < IBLK)
            def _():
                pltpu.async_copy(h_hbm.at[sidx_v.at[2 * k + 2]], r0, s0)

            pltpu.make_async_copy(h_hbm.at[sidx_v.at[0]], r1, s1).wait()
            pltpu.sync_copy(r1, acc.at[didx_v.at[c + 1]], add=True)
            return carry2

        lax.fori_loop(0, IBLK // 2, pair, 0)
        return carry

    lax.fori_loop(0, CPW // IBLK, block, 0)
    plsc.subcore_barrier()

    def wb(k, carry):
        r = tid * ROWS_PER_TILE + k * CHUNK
        pltpu.sync_copy(acc.at[pl.ds(r, CHUNK)], accp_hbm.at[cid, pl.ds(r, CHUNK)])
        return carry

    lax.fori_loop(0, ROWS_PER_TILE // CHUNK, wb, 0)


_agg = pl.kernel(
    _agg_body,
    out_type=jax.ShapeDtypeStruct((NC, N_PAD, D_IN), jnp.float32),
    mesh=_MESH,
    scratch_types=[
        pltpu.VMEM((IBLK, CHUNK), jnp.int32),
        pltpu.VMEM((CPW, CHUNK), jnp.int32),
        pltpu.VMEM((CHUNK, D_IN), jnp.float32),
        pltpu.VMEM((CHUNK, D_IN), jnp.float32),
        pltpu.VMEM_SHARED((N_PAD, D_IN), jnp.float32),
        pltpu.SemaphoreType.DMA,
        pltpu.SemaphoreType.DMA,
    ],
)


ROW_BLK = 1024


def _scale_body(x_ref, d_ref, o_ref):
    o_ref[...] = x_ref[...] * lax.rsqrt(jnp.maximum(d_ref[...], 1.0))


_scale = pl.pallas_call(
    _scale_body,
    out_shape=jax.ShapeDtypeStruct((N_PAD, D_IN), jnp.float32),
    grid=(N_PAD // ROW_BLK,),
    in_specs=[
        pl.BlockSpec((ROW_BLK, D_IN), lambda i: (i, 0)),
        pl.BlockSpec((ROW_BLK, 1), lambda i: (i, 0)),
    ],
    out_specs=pl.BlockSpec((ROW_BLK, D_IN), lambda i: (i, 0)),
)


HEAD_BLK = 1000


def _head_body(a_ref, d_ref, w1_ref, b1_ref, w2_ref, b2_ref, o_ref):
    a = a_ref[0] + a_ref[1]
    a = a * lax.rsqrt(jnp.maximum(d_ref[...], 1.0))
    h2 = jnp.dot(a, w1_ref[...], preferred_element_type=jnp.float32) + b1_ref[...]
    h2 = jnp.maximum(h2, 0.0)
    o_ref[...] = (
        jnp.dot(h2, w2_ref[...], preferred_element_type=jnp.float32) + b2_ref[...]
    )


_head = pl.pallas_call(
    _head_body,
    out_shape=jax.ShapeDtypeStruct((N, D_OUT), jnp.float32),
    grid=(N // HEAD_BLK,),
    in_specs=[
        pl.BlockSpec((NC, HEAD_BLK, D_IN), lambda i: (0, i, 0)),
        pl.BlockSpec((HEAD_BLK, 1), lambda i: (i, 0)),
        pl.BlockSpec((D_IN, D_IN), lambda i: (0, 0)),
        pl.BlockSpec((1, D_IN), lambda i: (0, 0)),
        pl.BlockSpec((D_IN, D_OUT), lambda i: (0, 0)),
        pl.BlockSpec((1, D_OUT), lambda i: (0, 0)),
    ],
    out_specs=pl.BlockSpec((HEAD_BLK, D_OUT), lambda i: (i, 0)),
)


@jax.jit
def kernel(n_feat, edge_index, W_gcn, b_gcn, W_lin, b_lin):
    pad = N + (jnp.arange(E_PAD - E, dtype=jnp.int32) % (N_PAD - N))
    ei = jnp.concatenate(
        [edge_index, jnp.broadcast_to(pad, (2, E_PAD - E))], axis=1
    ).reshape(2, NCH_PAD, CHUNK)
    x = jnp.concatenate(
        [n_feat, jnp.zeros((N_PAD - N, D_IN), jnp.float32)], axis=0
    )
    deg = _deg(ei)
    h = _scale(x, deg[0].reshape(N_PAD, 1))
    accp = _agg(h, ei)[:, :N]
    out = _head(
        accp,
        deg[1, :N].reshape(N, 1),
        W_gcn,
        b_gcn.reshape(1, D_IN),
        W_lin,
        b_lin.reshape(1, D_OUT),
    )
    return out


# speedup vs baseline: 3.3197x; 1.0534x over previous
import jax
import jax.numpy as jnp
from jax import lax
from jax.experimental import pallas as pl
from jax.experimental.pallas import tpu as pltpu
from jax.experimental.pallas import tpu_sc as plsc

N = 10000
E = 320000
D_IN = 128
D_OUT = 40

NC = 2
NS = 16
NW = NC * NS
L = 16

CH = 256
NCH = E // CH
PRE_D = NCH // NS + 1
CHA = 128
BLK = 20
EPB = BLK * CHA
NBLK = E // EPB
N_PAD = 10240
ROWS_PER_TILE = N_PAD // NS
ZR = 128

_MESH = plsc.VectorSubcoreMesh(
    core_axis_name="c", subcore_axis_name="s", num_cores=NC, num_subcores=NS
)


def _deg_body(edge_hbm, deg_hbm, idx_v, ones_v, zbuf_v, dacc, sem):
    cid = lax.axis_index("c")
    tid = lax.axis_index("s")
    s_ch = (tid * NCH) // NS
    cnt = ((tid + 1) * NCH) // NS - s_ch

    pltpu.async_copy(edge_hbm.at[cid, 0, pl.ds(s_ch * CH, PRE_D * CH)], idx_v, sem)

    def ones_loop(i, carry):
        ones_v[pl.ds(i * L, L)] = jnp.ones((L,), jnp.float32)
        return carry

    lax.fori_loop(0, CH // L, ones_loop, 0)

    def init_loop(i, carry):
        zbuf_v[pl.ds(i * L, L)] = jnp.zeros((L,), jnp.float32)
        return carry

    lax.fori_loop(0, ROWS_PER_TILE // L, init_loop, 0)

    pltpu.sync_copy(zbuf_v, dacc.at[pl.ds(tid * ROWS_PER_TILE, ROWS_PER_TILE)])
    pltpu.make_async_copy(
        edge_hbm.at[cid, 0, pl.ds(s_ch * CH, PRE_D * CH)], idx_v, sem
    ).wait()
    plsc.subcore_barrier()

    def body(j, carry):
        pltpu.sync_copy(ones_v, dacc.at[idx_v.at[pl.ds(j * CH, CH)]], add=True)
        return carry

    lax.fori_loop(0, cnt, body, 0)
    plsc.subcore_barrier()

    s = tid * ROWS_PER_TILE
    pltpu.sync_copy(
        dacc.at[pl.ds(s, ROWS_PER_TILE)],
        deg_hbm.at[cid, pl.ds(s, ROWS_PER_TILE)],
    )


_deg = pl.kernel(
    _deg_body,
    out_type=jax.ShapeDtypeStruct((NC, N_PAD), jnp.float32),
    mesh=_MESH,
    scratch_types=[
        pltpu.VMEM((PRE_D * CH,), jnp.int32),
        pltpu.VMEM((CH,), jnp.float32),
        pltpu.VMEM((ROWS_PER_TILE,), jnp.float32),
        pltpu.VMEM_SHARED((N_PAD,), jnp.float32),
        pltpu.SemaphoreType.DMA,
    ],
)


def _agg_body(h_hbm, edge_hbm, accp_hbm, sidx_v, didx_v, r0, r1, acc, s0, s1):
    cid = lax.axis_index("c")
    tid = lax.axis_index("s")
    wid = cid * NS + tid
    s_blk = (wid * NBLK) // NW
    nblk = ((wid + 1) * NBLK) // NW - s_blk

    def zloop(i, carry):
        r0[i // (D_IN // L), pl.ds((i % (D_IN // L)) * L, L)] = jnp.zeros(
            (L,), jnp.float32
        )
        return carry

    lax.fori_loop(0, ZR * (D_IN // L), zloop, 0)

    def zcopy(k, carry):
        pltpu.sync_copy(r0, acc.at[pl.ds(tid * ROWS_PER_TILE + k * ZR, ZR)])
        return carry

    lax.fori_loop(0, ROWS_PER_TILE // ZR, zcopy, 0)
    plsc.subcore_barrier()

    def block(ib, carry):
        gb = s_blk + ib
        pltpu.sync_copy(edge_hbm.at[0, 0, pl.ds(gb * EPB, EPB)], sidx_v)
        pltpu.sync_copy(edge_hbm.at[1, 0, pl.ds(gb * EPB, EPB)], didx_v)
        pltpu.async_copy(h_hbm.at[sidx_v.at[pl.ds(0, CHA)]], r0, s0)

        def pair(k, carry2):
            c = 2 * k
            pltpu.async_copy(h_hbm.at[sidx_v.at[pl.ds((c + 1) * CHA, CHA)]], r1, s1)
            pltpu.make_async_copy(h_hbm.at[sidx_v.at[pl.ds(0, CHA)]], r0, s0).wait()
            pltpu.sync_copy(r0, acc.at[didx_v.at[pl.ds(c * CHA, CHA)]], add=True)

            @pl.when(c + 2 < BLK)
            def _():
                pltpu.async_copy(
                    h_hbm.at[sidx_v.at[pl.ds((c + 2) * CHA, CHA)]], r0, s0
                )

            pltpu.make_async_copy(h_hbm.at[sidx_v.at[pl.ds(0, CHA)]], r1, s1).wait()
            pltpu.sync_copy(r1, acc.at[didx_v.at[pl.ds((c + 1) * CHA, CHA)]], add=True)
            return carry2

        lax.fori_loop(0, BLK // 2, pair, 0)
        return carry

    lax.fori_loop(0, nblk, block, 0)
    plsc.subcore_barrier()

    s = tid * ROWS_PER_TILE
    pltpu.sync_copy(
        acc.at[pl.ds(s, ROWS_PER_TILE)], accp_hbm.at[cid, pl.ds(s, ROWS_PER_TILE)]
    )


_agg = pl.kernel(
    _agg_body,
    out_type=jax.ShapeDtypeStruct((NC, N_PAD, D_IN), jnp.float32),
    mesh=_MESH,
    scratch_types=[
        pltpu.VMEM((EPB,), jnp.int32),
        pltpu.VMEM((EPB,), jnp.int32),
        pltpu.VMEM((ZR, D_IN), jnp.float32),
        pltpu.VMEM((ZR, D_IN), jnp.float32),
        pltpu.VMEM_SHARED((N_PAD, D_IN), jnp.float32),
        pltpu.SemaphoreType.DMA,
        pltpu.SemaphoreType.DMA,
    ],
)


ROW_BLK = 1000


def _scale_body(x_ref, d_ref, o_ref):
    o_ref[...] = x_ref[...] * lax.rsqrt(jnp.maximum(d_ref[...], 1.0))


_scale = pl.pallas_call(
    _scale_body,
    out_shape=jax.ShapeDtypeStruct((N, D_IN), jnp.float32),
    grid=(N // ROW_BLK,),
    in_specs=[
        pl.BlockSpec((ROW_BLK, D_IN), lambda i: (i, 0)),
        pl.BlockSpec((ROW_BLK, 1), lambda i: (i, 0)),
    ],
    out_specs=pl.BlockSpec((ROW_BLK, D_IN), lambda i: (i, 0)),
)


HEAD_BLK = 1000


def _head_body(a_ref, d_ref, w1_ref, b1_ref, w2_ref, b2_ref, o_ref):
    a = a_ref[0] + a_ref[1]
    a = a * lax.rsqrt(jnp.maximum(d_ref[...], 1.0))
    h2 = jnp.dot(a, w1_ref[...], preferred_element_type=jnp.float32) + b1_ref[...]
    h2 = jnp.maximum(h2, 0.0)
    o_ref[...] = (
        jnp.dot(h2, w2_ref[...], preferred_element_type=jnp.float32) + b2_ref[...]
    )


_head = pl.pallas_call(
    _head_body,
    out_shape=jax.ShapeDtypeStruct((N, D_OUT), jnp.float32),
    grid=(N // HEAD_BLK,),
    in_specs=[
        pl.BlockSpec((NC, HEAD_BLK, D_IN), lambda i: (0, i, 0)),
        pl.BlockSpec((HEAD_BLK, 1), lambda i: (i, 0)),
        pl.BlockSpec((D_IN, D_IN), lambda i: (0, 0)),
        pl.BlockSpec((1, D_IN), lambda i: (0, 0)),
        pl.BlockSpec((D_IN, D_OUT), lambda i: (0, 0)),
        pl.BlockSpec((1, D_OUT), lambda i: (0, 0)),
    ],
    out_specs=pl.BlockSpec((HEAD_BLK, D_OUT), lambda i: (i, 0)),
)


@jax.jit
def kernel(n_feat, edge_index, W_gcn, b_gcn, W_lin, b_lin):
    ei = edge_index.reshape(2, 1, E)
    deg = _deg(ei)
    h = _scale(n_feat, deg[0, :N].reshape(N, 1))
    accp = _agg(h, ei)[:, :N]
    out = _head(
        accp,
        deg[1, :N].reshape(N, 1),
        W_gcn,
        b_gcn.reshape(1, D_IN),
        W_lin,
        b_lin.reshape(1, D_OUT),
    )
    return out
